# baseline (speedup 1.0000x reference)
import jax
import jax.numpy as jnp
from jax import lax
from jax.experimental import pallas as pl
from jax.experimental.pallas import tpu as pltpu
from jax.experimental.pallas import tpu_sc as plsc

VOCAB = 1000000
D = 64
B = 4096
L = 200
LOUT = L + 2
CLS_ID = 1
PAD_ID = 0

NC, NS = 2, 16
NW = NC * NS
BPW = B // NW
NQ = 4
QSLOT = 104
QPOS = QSLOT // 2


def _start_gather(table_hbm, idx_all, rows, sem, i, q):
    pltpu.async_copy(
        table_hbm.at[idx_all.at[pl.ds(i * (NQ * QSLOT) + QSLOT * q, QSLOT)]],
        rows, sem)


def _drain_gather(table_hbm, rows, sem):
    pltpu.make_async_copy(table_hbm.at[pl.ds(0, QSLOT)], rows, sem).wait()


def _compute_quarter(rows, outb, q):
    npos = min(QPOS, L - QPOS * q)

    @plsc.parallel_loop(0, npos, unroll=4)
    def _(l):
        for j in range(D // 16):
            sl = pl.ds(16 * j, 16)
            outb[1 + QPOS * q + l, sl] = rows[2 * l, sl] + rows[2 * l + 1, sl]


def _body(idx_hbm, table_hbm, out_hbm, idx_all, rows0, rows1, outb0,
          cidx, gsem0, gsem1, osem0, csem):
    wid = lax.axis_index("s") * NC + lax.axis_index("c")
    base = wid * BPW
    rows = (rows0, rows1)
    gsem = (gsem0, gsem1)

    pltpu.sync_copy(idx_hbm.at[wid], idx_all)

    cidx[...] = jnp.where(lax.iota(jnp.int32, 16) < 1, CLS_ID, PAD_ID)
    pltpu.async_copy(table_hbm.at[cidx], rows0.at[pl.ds(0, 16)], csem).wait()
    for j in range(D // 16):
        sl = pl.ds(16 * j, 16)
        c = rows0[0, sl]
        p = rows0[1, sl]
        outb0[0, sl] = c + c
        outb0[LOUT - 1, sl] = p + p

    def _unit(par, i, q, nxt, k, guarded_wait):
        _drain_gather(table_hbm, rows[par], gsem[par])
        if nxt is not None:
            ni, nq, guarded = nxt
            if guarded:
                @pl.when(k < BPW // 2 - 1)
                def _():
                    _start_gather(table_hbm, idx_all, rows[1 - par],
                                  gsem[1 - par], ni, nq)
            else:
                _start_gather(table_hbm, idx_all, rows[1 - par],
                              gsem[1 - par], ni, nq)
        if q == 0:
            if guarded_wait:
                @pl.when(k >= 1)
                def _():
                    pltpu.make_async_copy(outb0, out_hbm.at[base],
                                          osem0).wait()
            else:
                pltpu.make_async_copy(outb0, out_hbm.at[base], osem0).wait()
        _compute_quarter(rows[par], outb0, q)
        if q == NQ - 1:
            pltpu.async_copy(outb0, out_hbm.at[base + i], osem0)

    _start_gather(table_hbm, idx_all, rows0, gsem0, 0, 0)

    def _iter(k, carry):
        i0 = 2 * k
        for u in range(2 * NQ):
            i = i0 + u // NQ
            q = u % NQ
            last = u == 2 * NQ - 1
            nxt = (i0 + (u + 1) // NQ, (u + 1) % NQ, last)
            _unit(u % 2, i, q, nxt, k, guarded_wait=(u == 0))
        return carry

    lax.fori_loop(0, BPW // 2, _iter, 0)

    pltpu.make_async_copy(outb0, out_hbm.at[base], osem0).wait()


@jax.jit
def kernel(unfold, emb_table):
    t128 = jnp.pad(emb_table, ((0, 0), (0, 64)))
    idx = unfold.astype(jnp.int32)[:, :, 0::2]
    idx = idx.reshape(B, 2 * L)
    idx = jnp.pad(idx, ((0, 0), (0, NQ * QSLOT - 2 * L)))
    idx = idx.reshape(NW, BPW * NQ * QSLOT)
    mesh = plsc.VectorSubcoreMesh(core_axis_name="c", subcore_axis_name="s",
                                  num_cores=NC, num_subcores=NS)
    run = pl.kernel(
        _body,
        out_type=jax.ShapeDtypeStruct((B, LOUT, D), jnp.float32),
        mesh=mesh,
        compiler_params=pltpu.CompilerParams(use_tc_tiling_on_sc=True),
        scratch_types=[
            pltpu.VMEM((BPW * NQ * QSLOT,), jnp.int32),
            pltpu.VMEM((QSLOT, 128), jnp.float32),
            pltpu.VMEM((QSLOT, 128), jnp.float32),
            pltpu.VMEM((LOUT, D), jnp.float32),
            pltpu.VMEM((16,), jnp.int32),
            pltpu.SemaphoreType.DMA,
            pltpu.SemaphoreType.DMA,
            pltpu.SemaphoreType.DMA,
            pltpu.SemaphoreType.DMA,
        ],
    )
    return run(idx, t128)


# speedup vs baseline: 3.2027x; 3.2027x over previous
import jax
import jax.numpy as jnp
from jax import lax
from jax.experimental import pallas as pl
from jax.experimental.pallas import tpu as pltpu
from jax.experimental.pallas import tpu_sc as plsc

VOCAB = 1000000
D = 64
B = 4096
L = 200
LOUT = L + 2
CLS_ID = 1
PAD_ID = 0

NC, NS = 2, 16
NW = NC * NS
BPW = B // NW
NCHUNK = 4
CHUNK = (2 * L) // NCHUNK


def _start_gathers(table_hbm, idx_all, rows, sem, i):
    for j in range(NCHUNK):
        pltpu.async_copy(table_hbm.at[idx_all.at[i, j]],
                         rows.at[pl.ds(j * CHUNK, CHUNK)], sem)


def _drain_gathers(table_hbm, rows, sem):
    pltpu.make_async_copy(table_hbm.at[pl.ds(0, 2 * L)], rows, sem).wait()


def _compute(rows, outb):
    @plsc.parallel_loop(0, L, unroll=4)
    def _(l):
        for j in range(D // 16):
            sl = pl.ds(16 * j, 16)
            outb[1 + l, sl] = rows[2 * l, sl] + rows[2 * l + 1, sl]


def _body(idx_hbm, table_hbm, out_hbm, idx_all, rows0, rows1, outb0, outb1,
          cidx, gsem0, gsem1, osem0, osem1, csem):
    wid = lax.axis_index("s") * NC + lax.axis_index("c")
    base = wid * BPW

    pltpu.sync_copy(idx_hbm.at[wid], idx_all)

    cidx[...] = jnp.where(lax.iota(jnp.int32, 16) < 1, CLS_ID, PAD_ID)
    pltpu.async_copy(table_hbm.at[cidx], rows0.at[pl.ds(0, 16)], csem).wait()
    for outb in (outb0, outb1):
        for j in range(D // 16):
            sl = pl.ds(16 * j, 16)
            c = rows0[0, sl]
            p = rows0[1, sl]
            outb[0, sl] = c + c
            outb[LOUT - 1, sl] = p + p

    _start_gathers(table_hbm, idx_all, rows0, gsem0, 0)

    def _steady(k, g_next0, g_next1, w_out0, w_out1):
        i0 = 2 * k
        _drain_gathers(table_hbm, rows0, gsem0)
        if g_next0:
            _start_gathers(table_hbm, idx_all, rows1, gsem1, i0 + 1)
        if w_out0:
            pltpu.make_async_copy(outb0, out_hbm.at[base], osem0).wait()
        _compute(rows0, outb0)
        pltpu.async_copy(outb0, out_hbm.at[base + i0], osem0)

        _drain_gathers(table_hbm, rows1, gsem1)
        if g_next1:
            _start_gathers(table_hbm, idx_all, rows0, gsem0, i0 + 2)
        if w_out1:
            pltpu.make_async_copy(outb1, out_hbm.at[base], osem1).wait()
        _compute(rows1, outb1)
        pltpu.async_copy(outb1, out_hbm.at[base + i0 + 1], osem1)

    _steady(0, True, True, False, False)

    def _loop_body(k, carry):
        _steady(k, True, True, True, True)
        return carry

    lax.fori_loop(1, BPW // 2 - 1, _loop_body, 0)

    _steady(BPW // 2 - 1, True, False, True, True)

    pltpu.make_async_copy(outb0, out_hbm.at[base], osem0).wait()
    pltpu.make_async_copy(outb1, out_hbm.at[base], osem1).wait()


@jax.jit
def kernel(unfold, emb_table):
    idx = unfold.astype(jnp.int32)[:, :, 0::2]
    idx = idx.reshape(NW, BPW, NCHUNK, CHUNK)
    mesh = plsc.VectorSubcoreMesh(core_axis_name="c", subcore_axis_name="s",
                                  num_cores=NC, num_subcores=NS)
    run = pl.kernel(
        _body,
        out_type=jax.ShapeDtypeStruct((B, LOUT, D), jnp.float32),
        mesh=mesh,
        compiler_params=pltpu.CompilerParams(use_tc_tiling_on_sc=False,
                                             needs_layout_passes=False),
        scratch_types=[
            pltpu.VMEM((BPW, NCHUNK, CHUNK), jnp.int32),
            pltpu.VMEM((2 * L, D), jnp.float32),
            pltpu.VMEM((2 * L, D), jnp.float32),
            pltpu.VMEM((LOUT, D), jnp.float32),
            pltpu.VMEM((LOUT, D), jnp.float32),
            pltpu.VMEM((16,), jnp.int32),
            pltpu.SemaphoreType.DMA,
            pltpu.SemaphoreType.DMA,
            pltpu.SemaphoreType.DMA,
            pltpu.SemaphoreType.DMA,
            pltpu.SemaphoreType.DMA,
        ],
    )
    return run(idx, emb_table)
